# Initial kernel scaffold; baseline (speedup 1.0000x reference)
import jax
import jax.numpy as jnp
from jax.experimental import pallas as pl


def kernel(x, edge_index, W_gcn, b_gcn, ggc_w, W_ih, W_hh, b_ih, b_hh, W1, b1):
    raise NotImplementedError("write your pallas kernel here")


# speedup vs baseline: 5.7732x; 5.7732x over previous
import functools

import jax
import jax.numpy as jnp
from jax import lax
from jax.experimental import pallas as pl
from jax.experimental.pallas import tpu as pltpu
from jax.experimental.pallas import tpu_sc as plsc

_N = 10000
_E = 320000
_D = 128
_H = 256
_NPAD = 10240
_ZS = _NPAD // 16
_CH = 128
_DUMP = _N
_BLK = 512
_NB = _NPAD // _BLK

@functools.cache
def _sc_kernels():
    mesh = plsc.VectorSubcoreMesh(core_axis_name="c", subcore_axis_name="s",
                                  num_cores=2, num_subcores=16)

    @functools.partial(
        pl.kernel,
        out_type=jax.ShapeDtypeStruct((2, _NPAD, 128), jnp.float32),
        mesh=mesh,
        scratch_types=[
            pltpu.VMEM((_CH,), jnp.int32),
            pltpu.VMEM((_CH,), jnp.int32),
            pltpu.VMEM((_CH, 128), jnp.float32),
            pltpu.VMEM_SHARED((_NPAD, 128), jnp.float32),
            pltpu.SemaphoreType.DMA,
        ],
    )
    def seg_sc(f2, gidx, didx, zrow, out, gbuf, dbuf, rbuf, acc, sem):
        c = lax.axis_index("c")
        s = lax.axis_index("s")
        n_chunks = gidx.shape[2]
        pltpu.sync_copy(zrow.at[pl.ds(s * _ZS, _ZS)], acc.at[pl.ds(s * _ZS, _ZS)])
        plsc.subcore_barrier()

        def body(j, carry):
            pltpu.sync_copy(gidx.at[c, s, j], gbuf)
            pltpu.sync_copy(didx.at[c, s, j], dbuf)
            pltpu.async_copy(f2.at[gbuf], rbuf, sem).wait()
            pltpu.sync_copy(rbuf, acc.at[dbuf], add=True)
            return carry

        lax.fori_loop(0, n_chunks, body, 0)
        plsc.subcore_barrier()
        pltpu.sync_copy(acc.at[pl.ds(s * _ZS, _ZS)], out.at[c, pl.ds(s * _ZS, _ZS)])

    @functools.partial(
        pl.kernel,
        out_type=jax.ShapeDtypeStruct((2, _NPAD, 128), jnp.float32),
        mesh=mesh,
        scratch_types=[
            pltpu.VMEM((_CH,), jnp.int32),
            pltpu.VMEM((_CH, 128), jnp.float32),
            pltpu.VMEM_SHARED((_NPAD, 128), jnp.float32),
        ],
    )
    def deg_sc(didx, zrow16, ones16, out, dbuf, onesbuf, acc):
        c = lax.axis_index("c")
        s = lax.axis_index("s")
        pltpu.sync_copy(zrow16.at[pl.ds(s * _ZS, _ZS)], acc.at[pl.ds(s * _ZS, _ZS)])
        pltpu.sync_copy(ones16, onesbuf)
        plsc.subcore_barrier()

        def body(j, carry):
            pltpu.sync_copy(didx.at[c, s, j], dbuf)
            pltpu.sync_copy(onesbuf, acc.at[dbuf], add=True)
            return carry

        lax.fori_loop(0, didx.shape[2], body, 0)
        plsc.subcore_barrier()
        pltpu.sync_copy(acc.at[pl.ds(s * _ZS, _ZS)], out.at[c, pl.ds(s * _ZS, _ZS)])

    return seg_sc, deg_sc


def _tc1_body(degp, xp, ggc, wih, y, dinv, wc):
    deg = degp[0, :, 0:1] + degp[1, :, 0:1] + 1.0
    dv = lax.rsqrt(deg)
    dinv[...] = jnp.broadcast_to(dv, (_NPAD, 128))
    y[...] = xp[...] * dv
    for i in range(3):
        wc[i, :, :] = lax.dot_general(
            ggc[i, :, :], wih[...], (((1,), (1,)), ((), ())),
            preferred_element_type=jnp.float32)


_tc1 = pl.pallas_call(
    _tc1_body,
    out_shape=[
        jax.ShapeDtypeStruct((_NPAD, 128), jnp.float32),
        jax.ShapeDtypeStruct((_NPAD, 128), jnp.float32),
        jax.ShapeDtypeStruct((3, 256, 768), jnp.float32),
    ],
)


def _leaky(v):
    return jnp.where(v >= 0, v, 0.01 * v)


def _tc2_body(p0, p1, y, dinv, wgcn, bgcn, whh, bhh, h, gh):
    sy = p0[...] + p1[...] + y[...]
    m = jnp.dot(sy, wgcn[...], preferred_element_type=jnp.float32)
    hh = _leaky(dinv[:, 0:1] * m + bgcn[...])
    h[...] = hh
    gh[...] = lax.dot_general(
        hh, whh[...], (((1,), (1,)), ((), ())),
        preferred_element_type=jnp.float32) + bhh[...]


_tc2 = pl.pallas_call(
    _tc2_body,
    grid=(_NB,),
    in_specs=[
        pl.BlockSpec((_BLK, 128), lambda i: (i, 0)),
        pl.BlockSpec((_BLK, 128), lambda i: (i, 0)),
        pl.BlockSpec((_BLK, 128), lambda i: (i, 0)),
        pl.BlockSpec((_BLK, 128), lambda i: (i, 0)),
        pl.BlockSpec((128, 256), lambda i: (0, 0)),
        pl.BlockSpec((1, 256), lambda i: (0, 0)),
        pl.BlockSpec((768, 256), lambda i: (0, 0)),
        pl.BlockSpec((1, 768), lambda i: (0, 0)),
    ],
    out_specs=[
        pl.BlockSpec((_BLK, 256), lambda i: (i, 0)),
        pl.BlockSpec((_BLK, 768), lambda i: (i, 0)),
    ],
    out_shape=[
        jax.ShapeDtypeStruct((_NPAD, 256), jnp.float32),
        jax.ShapeDtypeStruct((_NPAD, 768), jnp.float32),
    ],
)


def _gru_core(s0, s1, gh, xg, wci, bih):
    gi = (jnp.dot(s0[...], wci[0:128, :], preferred_element_type=jnp.float32)
          + jnp.dot(s1[...], wci[128:256, :], preferred_element_type=jnp.float32)
          + bih[...])
    ghv = gh[...]
    r = jax.nn.sigmoid(gi[:, 0:256] + ghv[:, 0:256])
    z = jax.nn.sigmoid(gi[:, 256:512] + ghv[:, 256:512])
    nn_ = jnp.tanh(gi[:, 512:768] + r * ghv[:, 512:768])
    return (1.0 - z) * nn_ + z * xg[...]


def _tc3_mid_body(s0, s1, gh, xg, wci, bih, whh, bhh, xg2, gh2):
    v = _gru_core(s0, s1, gh, xg, wci[...], bih)
    xg2[...] = v
    gh2[...] = lax.dot_general(
        v, whh[...], (((1,), (1,)), ((), ())),
        preferred_element_type=jnp.float32) + bhh[...]


_tc3_mid = pl.pallas_call(
    _tc3_mid_body,
    grid=(_NB,),
    in_specs=[
        pl.BlockSpec((_BLK, 128), lambda i: (i, 0)),
        pl.BlockSpec((_BLK, 128), lambda i: (i, 0)),
        pl.BlockSpec((_BLK, 768), lambda i: (i, 0)),
        pl.BlockSpec((_BLK, 256), lambda i: (i, 0)),
        pl.BlockSpec((256, 768), lambda i: (0, 0)),
        pl.BlockSpec((1, 768), lambda i: (0, 0)),
        pl.BlockSpec((768, 256), lambda i: (0, 0)),
        pl.BlockSpec((1, 768), lambda i: (0, 0)),
    ],
    out_specs=[
        pl.BlockSpec((_BLK, 256), lambda i: (i, 0)),
        pl.BlockSpec((_BLK, 768), lambda i: (i, 0)),
    ],
    out_shape=[
        jax.ShapeDtypeStruct((_NPAD, 256), jnp.float32),
        jax.ShapeDtypeStruct((_NPAD, 768), jnp.float32),
    ],
)


def _tc3_last_body(s0, s1, gh, xg, wci, bih, h, w1, b1, o):
    v = _gru_core(s0, s1, gh, xg, wci[...], bih)
    h2 = _leaky(v) + h[...]
    o[...] = jax.nn.sigmoid(
        jnp.dot(h2, w1[...], preferred_element_type=jnp.float32) + b1[...])


_tc3_last = pl.pallas_call(
    _tc3_last_body,
    grid=(_NB,),
    in_specs=[
        pl.BlockSpec((_BLK, 128), lambda i: (i, 0)),
        pl.BlockSpec((_BLK, 128), lambda i: (i, 0)),
        pl.BlockSpec((_BLK, 768), lambda i: (i, 0)),
        pl.BlockSpec((_BLK, 256), lambda i: (i, 0)),
        pl.BlockSpec((256, 768), lambda i: (0, 0)),
        pl.BlockSpec((1, 768), lambda i: (0, 0)),
        pl.BlockSpec((_BLK, 256), lambda i: (i, 0)),
        pl.BlockSpec((256, 256), lambda i: (0, 0)),
        pl.BlockSpec((1, 256), lambda i: (0, 0)),
    ],
    out_specs=[pl.BlockSpec((_BLK, 256), lambda i: (i, 0))],
    out_shape=[jax.ShapeDtypeStruct((_NPAD, 256), jnp.float32)],
)


def kernel(x, edge_index, W_gcn, b_gcn, ggc_w, W_ih, W_hh, b_ih, b_hh, W1, b1):
    src = edge_index[0].astype(jnp.int32)
    dst = edge_index[1].astype(jnp.int32)
    xp = jnp.pad(x, ((0, _NPAD - _N), (0, 0)))

    pt = _E // 32
    nca = -(-pt // _CH)
    pada = nca * _CH - pt
    src_a = src.reshape(2, 16, pt)
    dst_a = dst.reshape(2, 16, pt)
    gidx_a = jnp.pad(src_a, ((0, 0), (0, 0), (0, pada))).reshape(2, 16, nca, _CH)
    didx_a = jnp.pad(dst_a, ((0, 0), (0, 0), (0, pada)),
                     constant_values=_DUMP).reshape(2, 16, nca, _CH)

    ptb = _E // 16
    ncb = -(-ptb // _CH)
    padb = ncb * _CH - ptb
    s16 = src.reshape(16, ptb)
    d16 = dst.reshape(16, ptb)
    g2 = jnp.stack([2 * s16, 2 * s16 + 1])
    gidx_b = jnp.pad(g2, ((0, 0), (0, 0), (0, padb))).reshape(2, 16, ncb, _CH)
    didx_b = jnp.pad(jnp.stack([d16, d16]), ((0, 0), (0, 0), (0, padb)),
                     constant_values=_DUMP).reshape(2, 16, ncb, _CH)

    zrow = jnp.zeros((_NPAD, 128), jnp.float32)
    ones128 = jnp.ones((_CH, 128), jnp.float32)

    _seg_sc, _deg_sc = _sc_kernels()
    degp = _deg_sc(didx_a, zrow, ones128)
    y, dinv, wc = _tc1(degp, xp, ggc_w, W_ih)

    p = _seg_sc(y, gidx_a, didx_a, zrow)
    h, gh = _tc2(p[0], p[1], y, dinv, W_gcn, b_gcn.reshape(1, -1),
                 W_hh, b_hh.reshape(1, -1))

    xg = h
    for i in range(3):
        s = _seg_sc(xg.reshape(2 * _NPAD, 128), gidx_b, didx_b, zrow)
        if i < 2:
            xg, gh = _tc3_mid(s[0], s[1], gh, xg, wc[i], b_ih.reshape(1, -1),
                              W_hh, b_hh.reshape(1, -1))
        else:
            (out,) = _tc3_last(s[0], s[1], gh, xg, wc[i], b_ih.reshape(1, -1),
                               h, W1, b1.reshape(1, -1))
    return out[:_N]
